# baseline (speedup 1.0000x reference)
import functools

import jax
import jax.numpy as jnp
from jax import lax
from jax.experimental import pallas as pl
from jax.experimental.pallas import tpu as pltpu
from jax.experimental.pallas import tpu_sc as plsc

NC = 2
NS = 16
NW = NC * NS
LANES = 16

CHUNK = 48


def _sc_mesh():
    return plsc.VectorSubcoreMesh(
        core_axis_name="c", subcore_axis_name="s", num_cores=NC, num_subcores=NS
    )


def _sc_hist(dst, n_nodes, n_bins):
    (E,) = dst.shape
    e_per_w = E // NW
    assert E % NW == 0 and e_per_w % LANES == 0 and n_bins % LANES == 0

    @functools.partial(
        pl.kernel,
        out_type=jax.ShapeDtypeStruct((NW, n_bins), jnp.float32),
        mesh=_sc_mesh(),
        compiler_params=pltpu.CompilerParams(needs_layout_passes=False),
        scratch_types=[
            pltpu.VMEM((e_per_w,), jnp.int32),
            pltpu.VMEM((n_bins,), jnp.float32),
        ],
    )
    def hist_kernel(dst_hbm, out_hbm, dst_v, hist_v):
        wid = lax.axis_index("s") * NC + lax.axis_index("c")
        pltpu.sync_copy(dst_hbm.at[pl.ds(wid * e_per_w, e_per_w)], dst_v)
        zeros16 = jnp.zeros((LANES,), jnp.float32)

        def zero_body(i, _):
            hist_v[pl.ds(i * LANES, LANES)] = zeros16
            return 0

        lax.fori_loop(0, n_bins // LANES, zero_body, 0)
        ones16 = jnp.ones((LANES,), jnp.float32)

        def acc_body(i, _):
            idx = dst_v[pl.ds(i * LANES, LANES)]
            plsc.addupdate_scatter(hist_v, [idx], ones16)
            return 0

        lax.fori_loop(0, e_per_w // LANES, acc_body, 0)
        pltpu.sync_copy(hist_v, out_hbm.at[wid])

    return hist_kernel(dst)


NBUF = 4


def _sc_aggregate(y, src3, dst3, zeros_init):
    n_nodes, feat = y.shape
    cpw = src3.shape[1] // CHUNK
    n_acc = zeros_init.shape[0]
    rpt = n_acc // NS
    assert cpw % NBUF == 0 and n_acc % NS == 0 and rpt % 8 == 0

    @functools.partial(
        pl.kernel,
        out_type=jax.ShapeDtypeStruct((NC, n_acc, feat), jnp.float32),
        mesh=_sc_mesh(),
        compiler_params=pltpu.CompilerParams(needs_layout_passes=False),
        scratch_types=[
            pltpu.VMEM((cpw * CHUNK,), jnp.int32),
            pltpu.VMEM((cpw * CHUNK,), jnp.int32),
            pltpu.VMEM((NBUF, CHUNK, feat), jnp.float32),
            pltpu.VMEM_SHARED((n_acc, feat), jnp.float32),
            pltpu.SemaphoreType.DMA((NBUF,)),
            pltpu.SemaphoreType.DMA((NBUF,)),
        ],
    )
    def agg_kernel(y_hbm, src_hbm, dst_hbm, zero_hbm, out_hbm, src_v, dst_v,
                   rows_v, acc_sh, gsem, ssem):
        cid = lax.axis_index("c")
        sid = lax.axis_index("s")
        wid = sid * NC + cid

        pltpu.sync_copy(src_hbm.at[wid], src_v)
        pltpu.sync_copy(dst_hbm.at[wid], dst_v)
        pltpu.sync_copy(
            zero_hbm.at[pl.ds(sid * rpt, rpt)], acc_sh.at[pl.ds(sid * rpt, rpt)]
        )
        plsc.subcore_barrier()

        def start_gather(b, j):
            pltpu.async_copy(
                y_hbm.at[src_v.at[pl.ds(j * CHUNK, CHUNK)]], rows_v.at[b], gsem.at[b]
            )

        for b in range(NBUF):
            start_gather(b, b)

        def group_body(g, _):
            for b in range(NBUF):
                j = g * NBUF + b
                pltpu.make_async_copy(
                    y_hbm.at[pl.ds(0, CHUNK)], rows_v.at[b], gsem.at[b]
                ).wait()
                pltpu.sync_copy(
                    rows_v.at[b],
                    acc_sh.at[dst_v.at[pl.ds(j * CHUNK, CHUNK)]],
                    add=True,
                )

                @pl.when(g < cpw // NBUF - 1)
                def _():
                    start_gather(b, g * NBUF + b + NBUF)

            return 0

        lax.fori_loop(0, cpw // NBUF, group_body, 0)
        plsc.subcore_barrier()

        pltpu.sync_copy(
            acc_sh.at[pl.ds(sid * rpt, rpt)], out_hbm.at[cid, pl.ds(sid * rpt, rpt)]
        )

    return agg_kernel(y, src3, dst3, zeros_init)


def _tc_prescale(hist_t, x, w1):
    n_nodes, gene = x.shape
    hidden = w1.shape[1]
    blk = 2000
    nblk = n_nodes // blk
    assert n_nodes % blk == 0

    def body(hist_ref, x_ref, w1_ref, y1_ref, dinv_ref):
        deg = jnp.sum(hist_ref[...], axis=1, keepdims=True) + 1.0
        dinv = lax.rsqrt(deg)
        dinv_ref[...] = dinv
        y1_ref[...] = dinv * jnp.dot(
            x_ref[...], w1_ref[...], preferred_element_type=jnp.float32
        )

    return pl.pallas_call(
        body,
        grid=(nblk,),
        in_specs=[
            pl.BlockSpec((blk, NW), lambda i: (i, 0)),
            pl.BlockSpec((blk, gene), lambda i: (i, 0)),
            pl.BlockSpec((gene, hidden), lambda i: (0, 0)),
        ],
        out_specs=[
            pl.BlockSpec((blk, hidden), lambda i: (i, 0)),
            pl.BlockSpec((blk, 1), lambda i: (i, 0)),
        ],
        out_shape=[
            jax.ShapeDtypeStruct((n_nodes, hidden), jnp.float32),
            jax.ShapeDtypeStruct((n_nodes, 1), jnp.float32),
        ],
    )(hist_t, x, w1)


def _tc_mid(z1p, y1, dinv, b1, w2):
    n_nodes, hidden = y1.shape
    feat = w2.shape[1]
    blk = 2000
    nblk = n_nodes // blk

    def body(z_ref, y1_ref, dinv_ref, b1_ref, w2_ref, y2_ref):
        dinv = dinv_ref[...]
        s1 = jax.nn.relu(dinv * (z_ref[0] + z_ref[1] + y1_ref[...]) + b1_ref[...])
        y2_ref[...] = dinv * jnp.dot(
            s1, w2_ref[...], preferred_element_type=jnp.float32
        )

    return pl.pallas_call(
        body,
        grid=(nblk,),
        in_specs=[
            pl.BlockSpec((NC, blk, hidden), lambda i: (0, i, 0)),
            pl.BlockSpec((blk, hidden), lambda i: (i, 0)),
            pl.BlockSpec((blk, 1), lambda i: (i, 0)),
            pl.BlockSpec((1, hidden), lambda i: (0, 0)),
            pl.BlockSpec((hidden, feat), lambda i: (0, 0)),
        ],
        out_specs=pl.BlockSpec((blk, feat), lambda i: (i, 0)),
        out_shape=jax.ShapeDtypeStruct((n_nodes, feat), jnp.float32),
    )(z1p, y1, dinv, b1, w2)


def _tc_final(z2p, y2, dinv, b2, data_t):
    n_nodes, feat = y2.shape
    batch = data_t.shape[1]
    blk = 2000
    nblk = n_nodes // blk

    def body(z_ref, y2_ref, dinv_ref, b2_ref, data_ref, out_ref):
        h2 = dinv_ref[...] * (z_ref[0] + z_ref[1] + y2_ref[...]) + b2_ref[...]
        part = lax.dot_general(
            data_ref[...], h2, (((0,), (0,)), ((), ())),
            preferred_element_type=jnp.float32,
        )

        @pl.when(pl.program_id(0) == 0)
        def _():
            out_ref[...] = jnp.zeros_like(out_ref)

        out_ref[...] += part

    return pl.pallas_call(
        body,
        grid=(nblk,),
        in_specs=[
            pl.BlockSpec((NC, blk, feat), lambda i: (0, i, 0)),
            pl.BlockSpec((blk, feat), lambda i: (i, 0)),
            pl.BlockSpec((blk, 1), lambda i: (i, 0)),
            pl.BlockSpec((1, feat), lambda i: (0, 0)),
            pl.BlockSpec((blk, batch), lambda i: (i, 0)),
        ],
        out_specs=pl.BlockSpec((batch, feat), lambda i: (0, 0)),
        out_shape=jax.ShapeDtypeStruct((batch, feat), jnp.float32),
    )(z2p, y2, dinv, b2, data_t)


def kernel(data, x, edge_index, W1, b1, W2, b2):
    n_nodes = x.shape[0]
    E = edge_index.shape[1]

    cpw = -(-E // (NW * CHUNK) // NBUF) * NBUF
    e_pad = NW * cpw * CHUNK
    n_acc = -(-n_nodes // (8 * NS)) * (8 * NS)
    pad = e_pad - E
    src = jnp.concatenate([edge_index[0], jnp.zeros((pad,), jnp.int32)])
    pad_dst = n_nodes + (jnp.arange(pad, dtype=jnp.int32) % (n_acc - n_nodes))
    dst = jnp.concatenate([edge_index[1], pad_dst])
    src3 = src.reshape(NW, cpw * CHUNK)
    dst3 = dst.reshape(NW, cpw * CHUNK)

    zeros_init = jnp.zeros((n_acc, x.shape[1]), jnp.float32)
    hist = _sc_hist(dst, n_nodes, n_acc)
    hist_t = hist[:, :n_nodes].T

    y1, dinv = _tc_prescale(hist_t, x, W1)
    z1p = _sc_aggregate(y1, src3, dst3, zeros_init)
    y2 = _tc_mid(z1p, y1, dinv, b1.reshape(1, -1), W2)
    z2p = _sc_aggregate(y2, src3, dst3, zeros_init)
    return _tc_final(z2p, y2, dinv, b2.reshape(1, -1), data.T)
